# baseline (speedup 1.0000x reference)
import functools

import jax
import jax.numpy as jnp
from jax import lax
from jax.experimental import pallas as pl
from jax.experimental.pallas import tpu as pltpu
from jax.experimental.pallas import tpu_sc as plsc

R = 128
N = 32768
L = 16
NVEC = N // L
UNROLL = 16
CVEC = 16
CHUNK = CVEC * L
NCH = N // CHUNK
NC = 2
NS = 16
NW = NC * NS
ROWS_PER_W = R // NW
NBUF = 3


def _splat(x):
    return jnp.broadcast_to(x, (L,))


def _compute_row(row_v, cmax_sm, list_sm):
    def p1(c, gacc):
        a = row_v[pl.ds(c * CHUNK, L)]
        for k in range(1, CVEC):
            a = jnp.maximum(a, row_v[pl.ds(c * CHUNK + k * L, L)])
        cmax_sm[c] = jnp.max(a)
        return jnp.maximum(gacc, a)
    gacc = lax.fori_loop(0, NCH, p1, jnp.full((L,), -3.0e38, jnp.float32))
    rowmax = jnp.max(gacc)
    t0_s = rowmax - jnp.float32(1.0)
    t0 = _splat(t0_s)

    def p2(c, off):
        act = cmax_sm[c] > t0_s
        @pl.when(act)
        def _():
            list_sm[off] = c
        return off + jnp.where(act, jnp.int32(1), jnp.int32(0))
    nact = lax.fori_loop(0, NCH, p2, jnp.int32(0))

    def stats(t_v):
        t_s = jnp.max(t_v)
        def body(j, carry):
            ch = list_sm[j]
            def live(s, cnt):
                base = ch * CHUNK
                for k in range(CVEC):
                    v = row_v[pl.ds(base + k * L, L)]
                    m = v > t_v
                    s = s + jnp.where(m, v, jnp.float32(0.0))
                    cnt = cnt + jnp.where(m, jnp.float32(1.0),
                                          jnp.float32(0.0))
                return s, cnt
            return lax.cond(cmax_sm[ch] > t_s, live, lambda s, cnt: (s, cnt),
                            *carry)
        z = jnp.zeros((L,), jnp.float32)
        s, cnt = lax.fori_loop(0, nact, body, (z, z))
        return _splat(jnp.sum(s)), _splat(jnp.sum(cnt))

    s, c = stats(t0)
    t1 = (s - jnp.float32(1.0)) / c

    def cond(carry):
        t_prev, t_cur, it = carry
        return jnp.logical_and(jnp.all(t_cur > t_prev), it < jnp.int32(64))

    def wbody(carry):
        _, t_cur, it = carry
        s2, c2 = stats(t_cur)
        return t_cur, (s2 - jnp.float32(1.0)) / c2, it + jnp.int32(1)

    tau, _, _ = lax.while_loop(cond, wbody, (t0, t1, jnp.int32(0)))

    def mbody(i, carry):
        for j in range(UNROLL):
            sl = pl.ds((i * UNROLL + j) * L, L)
            v = row_v[sl]
            row_v[sl] = jnp.where(v > tau, jnp.float32(1.0), jnp.float32(0.0))
        return carry
    lax.fori_loop(0, NVEC // UNROLL, mbody, jnp.int32(0))


def _sc_body(x_hbm, out_hbm, row_v0, row_v1, row_v2, cmax_sm, list_sm,
             in_sem0, in_sem1, in_sem2, out_sem0, out_sem1, out_sem2):
    wid = lax.axis_index("s") * NC + lax.axis_index("c")
    base_row = wid * ROWS_PER_W
    bufs = (row_v0, row_v1, row_v2)
    in_sems = (in_sem0, in_sem1, in_sem2)
    out_sems = (out_sem0, out_sem1, out_sem2)

    in_h = {}
    out_h = {}
    in_h[0] = pltpu.async_copy(x_hbm.at[pl.ds(base_row * N, N)], bufs[0],
                               in_sems[0])
    for r in range(ROWS_PER_W):
        b = r % NBUF
        if r + 1 < ROWS_PER_W:
            nb = (r + 1) % NBUF
            if r + 1 >= NBUF:
                out_h[r + 1 - NBUF].wait()
            in_h[r + 1] = pltpu.async_copy(
                x_hbm.at[pl.ds((base_row + r + 1) * N, N)], bufs[nb],
                in_sems[nb])
        in_h[r].wait()
        _compute_row(bufs[b], cmax_sm, list_sm)
        out_h[r] = pltpu.async_copy(
            bufs[b], out_hbm.at[pl.ds((base_row + r) * N, N)], out_sems[b])
    for r in range(max(0, ROWS_PER_W - NBUF), ROWS_PER_W):
        out_h[r].wait()


_spardmax_sc = functools.partial(
    pl.kernel,
    out_type=jax.ShapeDtypeStruct((R * N,), jnp.float32),
    mesh=plsc.VectorSubcoreMesh(core_axis_name="c", subcore_axis_name="s"),
    scratch_types=[
        pltpu.VMEM((N,), jnp.float32),
        pltpu.VMEM((N,), jnp.float32),
        pltpu.VMEM((N,), jnp.float32),
        pltpu.SMEM((NCH,), jnp.float32),
        pltpu.SMEM((NCH,), jnp.int32),
        pltpu.SemaphoreType.DMA,
        pltpu.SemaphoreType.DMA,
        pltpu.SemaphoreType.DMA,
        pltpu.SemaphoreType.DMA,
        pltpu.SemaphoreType.DMA,
        pltpu.SemaphoreType.DMA,
    ],
    compiler_params=pltpu.CompilerParams(needs_layout_passes=False),
)(_sc_body)


def kernel(x):
    return _spardmax_sc(x.reshape(R * N)).reshape(R, N)


# speedup vs baseline: 1.5128x; 1.5128x over previous
import functools

import jax
import jax.numpy as jnp
from jax import lax
from jax.experimental import pallas as pl
from jax.experimental.pallas import tpu as pltpu
from jax.experimental.pallas import tpu_sc as plsc

R = 128
N = 32768
L = 16
NVEC = N // L
UNROLL = 16
CVEC = 16
CHUNK = CVEC * L
NCH = N // CHUNK
NC = 2
NS = 16
NW = NC * NS
ROWS_PER_W = R // NW
NBUF = 3


def _splat(x):
    return jnp.broadcast_to(x, (L,))


def _scan_row(row_v, cmax_sm, list_sm):
    def p1(c, gacc):
        a = row_v[pl.ds(c * CHUNK, L)]
        for k in range(1, CVEC):
            a = jnp.maximum(a, row_v[pl.ds(c * CHUNK + k * L, L)])
        cmax_sm[c] = jnp.max(a)
        return jnp.maximum(gacc, a)
    gacc = lax.fori_loop(0, NCH, p1, jnp.full((L,), -3.0e38, jnp.float32))
    rowmax = jnp.max(gacc)
    t0_s = rowmax - jnp.float32(1.0)
    t0 = _splat(t0_s)

    def p2(c, off):
        act = cmax_sm[c] > t0_s
        @pl.when(act)
        def _():
            list_sm[off] = c
        return off + jnp.where(act, jnp.int32(1), jnp.int32(0))
    nact = lax.fori_loop(0, NCH, p2, jnp.int32(0))

    def stats(t_v):
        t_s = jnp.max(t_v)
        def body(j, carry):
            ch = list_sm[j]
            def live(s, cnt):
                base = ch * CHUNK
                for k in range(CVEC):
                    v = row_v[pl.ds(base + k * L, L)]
                    m = v > t_v
                    s = s + jnp.where(m, v, jnp.float32(0.0))
                    cnt = cnt + jnp.where(m, jnp.float32(1.0),
                                          jnp.float32(0.0))
                return s, cnt
            return lax.cond(cmax_sm[ch] > t_s, live, lambda s, cnt: (s, cnt),
                            *carry)
        z = jnp.zeros((L,), jnp.float32)
        s, cnt = lax.fori_loop(0, nact, body, (z, z))
        return _splat(jnp.sum(s)), _splat(jnp.sum(cnt))

    s, c = stats(t0)
    t1 = (s - jnp.float32(1.0)) / c

    def cond(carry):
        t_prev, t_cur, it = carry
        return jnp.logical_and(jnp.all(t_cur > t_prev), it < jnp.int32(64))

    def wbody(carry):
        _, t_cur, it = carry
        s2, c2 = stats(t_cur)
        return t_cur, (s2 - jnp.float32(1.0)) / c2, it + jnp.int32(1)

    tau, _, _ = lax.while_loop(cond, wbody, (t0, t1, jnp.int32(0)))
    return tau


def _mask_row(row_v, tau):
    def mbody(i, carry):
        for j in range(UNROLL):
            sl = pl.ds((i * UNROLL + j) * L, L)
            v = row_v[sl]
            row_v[sl] = jnp.where(v > tau, jnp.float32(1.0), jnp.float32(0.0))
        return carry
    lax.fori_loop(0, NVEC // UNROLL, mbody, jnp.int32(0))


def _sc_body(x_hbm, out_hbm, row_v0, row_v1, row_v2, sh_in,
             cmax_sm, list_sm, sem_a, sem_b, out_sem0, out_sem1, out_sem2):
    sid = lax.axis_index("s")
    wid = sid * NC + lax.axis_index("c")
    base = wid * ROWS_PER_W
    bufs = (row_v0, row_v1, row_v2)
    out_sems = (out_sem0, out_sem1, out_sem2)
    RPW = ROWS_PER_W

    def hbm_in(r):
        return pltpu.async_copy(x_hbm.at[base + r], sh_in.at[sid], sem_a)

    def to_tile(r):
        return pltpu.async_copy(sh_in.at[sid], bufs[r % NBUF], sem_b)

    def hbm_out(r):
        return pltpu.async_copy(bufs[r % NBUF], out_hbm.at[base + r],
                                out_sems[r % NBUF])

    hbm_in(0).wait()
    to_tile(0).wait()
    a_h = hbm_in(1)
    b_h = None
    out_h = {}
    for r in range(RPW):
        tau = _scan_row(bufs[r % NBUF], cmax_sm, list_sm)
        if r + 1 < RPW:
            if r - 2 >= 0:
                out_h[r - 2].wait()
            a_h.wait()
            b_h = to_tile(r + 1)
        _mask_row(bufs[r % NBUF], tau)
        if r + 2 < RPW:
            b_h.wait()
            a_h = hbm_in(r + 2)
        elif r + 1 < RPW:
            b_h.wait()
        out_h[r] = hbm_out(r)
    for r in range(max(0, RPW - NBUF), RPW):
        out_h[r].wait()


_spardmax_sc = functools.partial(
    pl.kernel,
    out_type=jax.ShapeDtypeStruct((R, N), jnp.float32),
    mesh=plsc.VectorSubcoreMesh(core_axis_name="c", subcore_axis_name="s"),
    scratch_types=[
        pltpu.VMEM((N,), jnp.float32),
        pltpu.VMEM((N,), jnp.float32),
        pltpu.VMEM((N,), jnp.float32),
        pltpu.VMEM_SHARED((NS, N), jnp.float32),
        pltpu.SMEM((NCH,), jnp.float32),
        pltpu.SMEM((NCH,), jnp.int32),
        pltpu.SemaphoreType.DMA,
        pltpu.SemaphoreType.DMA,
        pltpu.SemaphoreType.DMA,
        pltpu.SemaphoreType.DMA,
        pltpu.SemaphoreType.DMA,
    ],
    compiler_params=pltpu.CompilerParams(needs_layout_passes=False),
)(_sc_body)


def kernel(x):
    return _spardmax_sc(x)


# speedup vs baseline: 1.7261x; 1.1410x over previous
import functools

import jax
import jax.numpy as jnp
from jax import lax
from jax.experimental import pallas as pl
from jax.experimental.pallas import tpu as pltpu
from jax.experimental.pallas import tpu_sc as plsc

R = 128
N = 32768
L = 16
NVEC = N // L
UNROLL = 16
CVEC = 16
CHUNK = CVEC * L
NCH = N // CHUNK
NC = 2
NS = 16
NW = NC * NS
ROWS_PER_W = R // NW
NBUF = 3


def _splat(x):
    return jnp.broadcast_to(x, (L,))


def _scan_row(row_v, cmax_sm, list_sm, list2_sm):
    def p1(c, gacc):
        a = row_v[pl.ds(c * CHUNK, L)]
        for k in range(1, CVEC):
            a = jnp.maximum(a, row_v[pl.ds(c * CHUNK + k * L, L)])
        cmax_sm[c] = jnp.max(a)
        return jnp.maximum(gacc, a)
    gacc = lax.fori_loop(0, NCH, p1, jnp.full((L,), -3.0e38, jnp.float32))
    rowmax = jnp.max(gacc)
    t0_s = rowmax - jnp.float32(1.0)
    t0 = _splat(t0_s)

    def p2(c, off):
        act = cmax_sm[c] > t0_s
        @pl.when(act)
        def _():
            list_sm[off] = c
        return off + jnp.where(act, jnp.int32(1), jnp.int32(0))
    nact = lax.fori_loop(0, NCH, p2, jnp.int32(0))

    def stats(lst, t_v, m):
        def body(j, carry):
            s, cnt = carry
            base = lst[j] * CHUNK
            for k in range(CVEC):
                v = row_v[pl.ds(base + k * L, L)]
                msk = v > t_v
                s = s + jnp.where(msk, v, jnp.float32(0.0))
                cnt = cnt + jnp.where(msk, jnp.float32(1.0), jnp.float32(0.0))
            return s, cnt
        z = jnp.zeros((L,), jnp.float32)
        s, cnt = lax.fori_loop(0, m, body, (z, z))
        return _splat(jnp.sum(s)), _splat(jnp.sum(cnt))

    s, c = stats(list_sm, t0, nact)
    t1_v = (s - jnp.float32(1.0)) / c
    t1_s = jnp.max(t1_v)

    def rb(j, off):
        ch = list_sm[j]
        act = cmax_sm[ch] > t1_s
        @pl.when(act)
        def _():
            list2_sm[off] = ch
        return off + jnp.where(act, jnp.int32(1), jnp.int32(0))
    m2 = lax.fori_loop(0, nact, rb, jnp.int32(0))

    def cond(carry):
        t_prev, t_cur, it = carry
        return jnp.logical_and(jnp.all(t_cur > t_prev), it < jnp.int32(64))

    def wbody(carry):
        _, t_cur, it = carry
        s2, c2 = stats(list2_sm, t_cur, m2)
        return t_cur, (s2 - jnp.float32(1.0)) / c2, it + jnp.int32(1)

    tau, _, _ = lax.while_loop(cond, wbody, (t0, t1_v, jnp.int32(0)))
    return tau


def _mask_row(row_v, tau):
    def mbody(i, carry):
        for j in range(UNROLL):
            sl = pl.ds((i * UNROLL + j) * L, L)
            v = row_v[sl]
            row_v[sl] = jnp.where(v > tau, jnp.float32(1.0), jnp.float32(0.0))
        return carry
    lax.fori_loop(0, NVEC // UNROLL, mbody, jnp.int32(0))


def _sc_body(x_hbm, out_hbm, row_v0, row_v1, row_v2, sh_in,
             cmax_sm, list_sm, list2_sm, sem_a, sem_b,
             out_sem0, out_sem1, out_sem2):
    sid = lax.axis_index("s")
    wid = sid * NC + lax.axis_index("c")
    base = wid * ROWS_PER_W
    bufs = (row_v0, row_v1, row_v2)
    out_sems = (out_sem0, out_sem1, out_sem2)
    RPW = ROWS_PER_W

    def hbm_in(r):
        return pltpu.async_copy(x_hbm.at[base + r], sh_in.at[sid], sem_a)

    def to_tile(r):
        return pltpu.async_copy(sh_in.at[sid], bufs[r % NBUF], sem_b)

    def hbm_out(r):
        return pltpu.async_copy(bufs[r % NBUF], out_hbm.at[base + r],
                                out_sems[r % NBUF])

    hbm_in(0).wait()
    to_tile(0).wait()
    a_h = hbm_in(1)
    b_h = None
    out_h = {}
    for r in range(RPW):
        with jax.named_scope(f"scan{r}"):
            tau = _scan_row(bufs[r % NBUF], cmax_sm, list_sm, list2_sm)
        if r + 1 < RPW:
            with jax.named_scope(f"wOut{r}"):
                if r - 2 >= 0:
                    out_h[r - 2].wait()
            with jax.named_scope(f"wA{r}"):
                a_h.wait()
            b_h = to_tile(r + 1)
        with jax.named_scope(f"mask{r}"):
            _mask_row(bufs[r % NBUF], tau)
        if r + 2 < RPW:
            with jax.named_scope(f"wB{r}"):
                b_h.wait()
            a_h = hbm_in(r + 2)
        elif r + 1 < RPW:
            with jax.named_scope(f"wB{r}"):
                b_h.wait()
        out_h[r] = hbm_out(r)
    with jax.named_scope("drain"):
        for r in range(max(0, RPW - NBUF), RPW):
            out_h[r].wait()


_spardmax_sc = functools.partial(
    pl.kernel,
    out_type=jax.ShapeDtypeStruct((R, N), jnp.float32),
    mesh=plsc.VectorSubcoreMesh(core_axis_name="c", subcore_axis_name="s"),
    scratch_types=[
        pltpu.VMEM((N,), jnp.float32),
        pltpu.VMEM((N,), jnp.float32),
        pltpu.VMEM((N,), jnp.float32),
        pltpu.VMEM_SHARED((NS, N), jnp.float32),
        pltpu.SMEM((NCH,), jnp.float32),
        pltpu.SMEM((NCH,), jnp.int32),
        pltpu.SMEM((NCH,), jnp.int32),
        pltpu.SemaphoreType.DMA,
        pltpu.SemaphoreType.DMA,
        pltpu.SemaphoreType.DMA,
        pltpu.SemaphoreType.DMA,
        pltpu.SemaphoreType.DMA,
    ],
    compiler_params=pltpu.CompilerParams(needs_layout_passes=False),
)(_sc_body)


def kernel(x):
    return _spardmax_sc(x)
